# baseline (speedup 1.0000x reference)
import functools

import jax
import jax.numpy as jnp
from jax import lax
from jax.experimental import pallas as pl
from jax.experimental.pallas import tpu as pltpu
from jax.experimental.pallas import tpu_sc as plsc

_VOCAB = 100000
_D = 64
_B = 4096
_S = 200
_LN = 128
_NC = 2
_NS = 16
_NW = _NC * _NS
_TC = _B // _LN

_mesh = plsc.VectorSubcoreMesh(core_axis_name="c", subcore_axis_name="s")


@functools.partial(
    pl.kernel,
    out_type=jax.ShapeDtypeStruct((_S, _D // 8, _TC, 8, _LN), jnp.float32),
    mesh=_mesh,
    scratch_types=[
        pltpu.VMEM((_S, _LN), jnp.int32),
        pltpu.VMEM((2, _LN, _D), jnp.float32),
        pltpu.VMEM((2, _D, _LN), jnp.float32),
    ]
    + [pltpu.SemaphoreType.DMA] * 4,
    compiler_params=pltpu.CompilerParams(use_tc_tiling_on_sc=False, needs_layout_passes=False),
)
def _sc_gather(table_hbm, ids_hbm, out_hbm, idx_all, raw_v, tb_v, g0, g1, w0, w1):
    gs = (g0, g1)
    ws = (w0, w1)
    tc = lax.axis_index("s") * _NC + lax.axis_index("c")

    pltpu.sync_copy(ids_hbm.at[:, pl.ds(tc * _LN, _LN)], idx_all)

    def fire_gather(s, b):
        pltpu.make_async_copy(
            table_hbm.at[idx_all.at[s]], raw_v.at[b], gs[b]
        ).start()

    def wait_gather(b):
        pltpu.make_async_copy(
            table_hbm.at[pl.ds(0, _LN)], raw_v.at[b], gs[b]
        ).wait()

    def fire_writes(s, b):
        for tr in range(_D // 8):
            pltpu.make_async_copy(
                tb_v.at[b, pl.ds(tr * 8, 8)], out_hbm.at[s, tr, tc], ws[b]
            ).start()

    def wait_writes(b):
        for tr in range(_D // 8):
            pltpu.make_async_copy(
                tb_v.at[b, pl.ds(tr * 8, 8)], out_hbm.at[0, tr, 0], ws[b]
            ).wait()

    iotas = [lax.iota(jnp.int32, 16) + 16 * q for q in range(4)]

    def transpose(b):
        def row8(r8, carry):
            for rr in range(8):
                r = r8 * 8 + rr
                colv = jnp.zeros((16,), jnp.int32) + r
                for q in range(4):
                    v = raw_v[b, r, pl.ds(16 * q, 16)]
                    plsc.store_scatter(tb_v.at[b], [iotas[q], colv], v)
            return carry

        lax.fori_loop(0, _LN // 8, row8, 0)

    fire_gather(0, 0)
    fire_gather(1, 1)
    for s in (0, 1):
        wait_gather(s)
        transpose(s)
        fire_gather(s + 2, s)
        fire_writes(s, s)

    def sbody(t, carry):
        for u in range(2):
            s = 2 + 2 * t + u
            b = u
            wait_gather(b)
            wait_writes(b)
            transpose(b)

            @pl.when(s + 2 < _S)
            def _():
                fire_gather(s + 2, b)

            fire_writes(s, b)
        return carry

    lax.fori_loop(0, (_S - 2) // 2, sbody, 0)
    wait_writes(0)
    wait_writes(1)


def kernel(ipa_ids, lengths, table):
    ids_t = ipa_ids.T
    out5 = _sc_gather(table, ids_t)
    out = lax.reshape(out5, (_B, _S, _D), dimensions=(2, 4, 0, 1, 3))
    return (out, lengths)


# speedup vs baseline: 2.3201x; 2.3201x over previous
import functools

import jax
import jax.numpy as jnp
from jax import lax
from jax.experimental import pallas as pl
from jax.experimental.pallas import tpu as pltpu
from jax.experimental.pallas import tpu_sc as plsc

_VOCAB = 100000
_D = 64
_B = 4096
_S = 200
_LN = 128
_NC = 2
_NS = 16
_NW = _NC * _NS
_TC = _B // _LN

_mesh = plsc.VectorSubcoreMesh(core_axis_name="c", subcore_axis_name="s")


@functools.partial(
    pl.kernel,
    out_type=jax.ShapeDtypeStruct((_S, _D // 8, _TC, 8, _LN), jnp.float32),
    mesh=_mesh,
    scratch_types=[
        pltpu.VMEM((_S, _LN), jnp.int32),
        pltpu.VMEM((2, _LN, _D), jnp.float32),
        pltpu.VMEM((2, _D, _LN + 1), jnp.float32),
    ]
    + [pltpu.SemaphoreType.DMA] * 4,
    compiler_params=pltpu.CompilerParams(use_tc_tiling_on_sc=False, needs_layout_passes=False),
)
def _sc_gather(table_hbm, ids_hbm, out_hbm, idx_all, raw_v, tb_v, g0, g1, w0, w1):
    gs = (g0, g1)
    ws = (w0, w1)
    tc = lax.axis_index("s") * _NC + lax.axis_index("c")

    pltpu.sync_copy(ids_hbm.at[:, pl.ds(tc * _LN, _LN)], idx_all)

    def fire_gather(s, b):
        pltpu.make_async_copy(
            table_hbm.at[idx_all.at[s]], raw_v.at[b], gs[b]
        ).start()

    def wait_gather(b):
        pltpu.make_async_copy(
            table_hbm.at[pl.ds(0, _LN)], raw_v.at[b], gs[b]
        ).wait()

    def fire_writes(s, b):
        for tr in range(_D // 8):
            pltpu.make_async_copy(
                tb_v.at[b, pl.ds(tr * 8, 8), pl.ds(0, _LN)],
                out_hbm.at[s, tr, tc], ws[b]
            ).start()

    def wait_writes(b):
        for tr in range(_D // 8):
            pltpu.make_async_copy(
                tb_v.at[b, pl.ds(tr * 8, 8), pl.ds(0, _LN)],
                out_hbm.at[0, tr, 0], ws[b]
            ).wait()

    iotas = [lax.iota(jnp.int32, 16) + 16 * q for q in range(4)]

    def transpose(b):
        def row8(r8, carry):
            for rr in range(8):
                r = r8 * 8 + rr
                colv = jnp.zeros((16,), jnp.int32) + r
                for q in range(4):
                    v = raw_v[b, r, pl.ds(16 * q, 16)]
                    plsc.store_scatter(tb_v.at[b], [iotas[q], colv], v)
            return carry

        lax.fori_loop(0, _LN // 8, row8, 0)

    fire_gather(0, 0)
    fire_gather(1, 1)
    for s in (0, 1):
        wait_gather(s)
        transpose(s)
        fire_gather(s + 2, s)
        fire_writes(s, s)

    def sbody(t, carry):
        for u in range(2):
            s = 2 + 2 * t + u
            b = u
            wait_gather(b)
            wait_writes(b)
            transpose(b)

            @pl.when(s + 2 < _S)
            def _():
                fire_gather(s + 2, b)

            fire_writes(s, b)
        return carry

    lax.fori_loop(0, (_S - 2) // 2, sbody, 0)
    wait_writes(0)
    wait_writes(1)


def kernel(ipa_ids, lengths, table):
    ids_t = ipa_ids.T
    out5 = _sc_gather(table, ids_t)
    out = lax.reshape(out5, (_B, _S, _D), dimensions=(2, 4, 0, 1, 3))
    return (out, lengths)


# speedup vs baseline: 2.6183x; 1.1285x over previous
import functools

import jax
import jax.numpy as jnp
from jax import lax
from jax.experimental import pallas as pl
from jax.experimental.pallas import tpu as pltpu
from jax.experimental.pallas import tpu_sc as plsc

_VOCAB = 100000
_D = 64
_B = 4096
_S = 200
_LN = 128
_NC = 2
_NS = 16
_NW = _NC * _NS
_TC = _B // _LN
_PITCH = _LN + 1

_mesh = plsc.VectorSubcoreMesh(core_axis_name="c", subcore_axis_name="s")


@functools.partial(
    pl.kernel,
    out_type=jax.ShapeDtypeStruct((_S, _D // 8, _TC, 8, _LN), jnp.float32),
    mesh=_mesh,
    scratch_types=[
        pltpu.VMEM((_S, _LN), jnp.int32),
        pltpu.VMEM((4, _LN, _D), jnp.float32),
        pltpu.VMEM((2, _D, _PITCH), jnp.float32),
    ]
    + [pltpu.SemaphoreType.DMA] * 6,
    compiler_params=pltpu.CompilerParams(
        use_tc_tiling_on_sc=False, needs_layout_passes=False
    ),
)
def _sc_gather(table_hbm, ids_hbm, out_hbm, idx_all, raw_v, tb_v,
               g0, g1, g2, g3, w0, w1):
    gs = (g0, g1, g2, g3)
    ws = (w0, w1)
    tc = lax.axis_index("s") * _NC + lax.axis_index("c")

    pltpu.sync_copy(ids_hbm.at[:, pl.ds(tc * _LN, _LN)], idx_all)

    def fire_gather(s, br):
        pltpu.make_async_copy(
            table_hbm.at[idx_all.at[s]], raw_v.at[br], gs[br]
        ).start()

    def wait_gather(br):
        pltpu.make_async_copy(
            table_hbm.at[pl.ds(0, _LN)], raw_v.at[br], gs[br]
        ).wait()

    def fire_writes(s, bt):
        for tr in range(_D // 8):
            pltpu.make_async_copy(
                tb_v.at[bt, pl.ds(tr * 8, 8), pl.ds(0, _LN)],
                out_hbm.at[s, tr, tc], ws[bt]
            ).start()

    def wait_writes(bt):
        for tr in range(_D // 8):
            pltpu.make_async_copy(
                tb_v.at[bt, pl.ds(tr * 8, 8), pl.ds(0, _LN)],
                out_hbm.at[0, tr, 0], ws[bt]
            ).wait()

    iotas = [lax.iota(jnp.int32, 16) + 16 * q for q in range(4)]
    zeros16 = jnp.zeros((16,), jnp.int32)

    def transpose(br, bt):
        def grp(g16, carry):
            r0 = g16 * 16
            for rr in range(16):
                r = r0 + rr
                colv = zeros16 + r
                vs = [raw_v[br, r, pl.ds(16 * q, 16)] for q in range(4)]
                for q in range(4):
                    plsc.store_scatter(tb_v.at[bt], [iotas[q], colv], vs[q])
            return carry

        lax.fori_loop(0, _LN // 16, grp, 0)

    for s in range(4):
        fire_gather(s, s)
    for s in range(4):
        wait_gather(s)
        if s >= 2:
            wait_writes(s % 2)
        transpose(s, s % 2)
        fire_gather(s + 4, s)
        fire_writes(s, s % 2)

    def sbody(t, carry):
        for u in range(4):
            s = 4 + 4 * t + u
            br = u
            bt = u % 2
            wait_gather(br)
            wait_writes(bt)
            transpose(br, bt)

            @pl.when(s + 4 < _S)
            def _():
                fire_gather(s + 4, br)

            fire_writes(s, bt)
        return carry

    lax.fori_loop(0, (_S - 4) // 4, sbody, 0)
    wait_writes(0)
    wait_writes(1)


def kernel(ipa_ids, lengths, table):
    ids_t = ipa_ids.T
    out5 = _sc_gather(table, ids_t)
    out = lax.reshape(out5, (_B, _S, _D), dimensions=(2, 4, 0, 1, 3))
    return (out, lengths)


# speedup vs baseline: 4.0888x; 1.5616x over previous
import functools

import jax
import jax.numpy as jnp
from jax import lax
from jax.experimental import pallas as pl
from jax.experimental.pallas import tpu as pltpu
from jax.experimental.pallas import tpu_sc as plsc

_VOCAB = 100000
_D = 64
_B = 4096
_S = 200
_LN = 128
_NC = 2
_NS = 16
_NW = _NC * _NS
_TC = _B // _LN
_PITCH = _LN + 1

_mesh = plsc.VectorSubcoreMesh(core_axis_name="c", subcore_axis_name="s")


@functools.partial(
    pl.kernel,
    out_type=jax.ShapeDtypeStruct((_S, _D // 8, _TC, 8, _LN), jnp.float32),
    mesh=_mesh,
    scratch_types=[
        pltpu.VMEM((_S, _LN), jnp.int32),
        pltpu.VMEM((4, _LN, _D), jnp.float32),
        pltpu.VMEM((2, _D, _PITCH), jnp.float32),
    ]
    + [pltpu.SemaphoreType.DMA] * 6,
    compiler_params=pltpu.CompilerParams(
        use_tc_tiling_on_sc=False, needs_layout_passes=False
    ),
)
def _sc_gather(table_hbm, ids_hbm, out_hbm, idx_all, raw_v, tb_v,
               g0, g1, g2, g3, w0, w1):
    gs = (g0, g1, g2, g3)
    ws = (w0, w1)
    tc = lax.axis_index("s") * _NC + lax.axis_index("c")

    pltpu.sync_copy(ids_hbm.at[:, pl.ds(tc * _LN, _LN)], idx_all)

    def fire_gather(s, br):
        pltpu.make_async_copy(
            table_hbm.at[idx_all.at[s]], raw_v.at[br], gs[br]
        ).start()

    def wait_gather(br):
        pltpu.make_async_copy(
            table_hbm.at[pl.ds(0, _LN)], raw_v.at[br], gs[br]
        ).wait()

    def fire_writes(s, bt):
        for tr in range(_D // 8):
            pltpu.make_async_copy(
                tb_v.at[bt, pl.ds(tr * 8, 8), pl.ds(0, _LN)],
                out_hbm.at[s, tr, tc], ws[bt]
            ).start()

    def wait_writes(bt):
        for tr in range(_D // 8):
            pltpu.make_async_copy(
                tb_v.at[bt, pl.ds(tr * 8, 8), pl.ds(0, _LN)],
                out_hbm.at[0, tr, 0], ws[bt]
            ).wait()

    iotas = [lax.iota(jnp.int32, 16) + 16 * q for q in range(4)]
    zeros16 = jnp.zeros((16,), jnp.int32)

    def transpose(br, bt):
        @plsc.parallel_loop(0, _LN // 4, unroll=4)
        def grp(g4):
            r0 = g4 * 4
            for rr in range(4):
                r = r0 + rr
                colv = zeros16 + r
                vs = [raw_v[br, r, pl.ds(16 * q, 16)] for q in range(4)]
                for q in range(4):
                    plsc.store_scatter(tb_v.at[bt], [iotas[q], colv], vs[q])

    for s in range(4):
        fire_gather(s, s)
    for s in range(4):
        wait_gather(s)
        if s >= 2:
            wait_writes(s % 2)
        transpose(s, s % 2)
        fire_gather(s + 4, s)
        fire_writes(s, s % 2)

    def sbody(t, carry):
        for u in range(4):
            s = 4 + 4 * t + u
            br = u
            bt = u % 2
            wait_gather(br)
            wait_writes(bt)
            transpose(br, bt)

            @pl.when(s + 4 < _S)
            def _():
                fire_gather(s + 4, br)

            fire_writes(s, bt)
        return carry

    lax.fori_loop(0, (_S - 4) // 4, sbody, 0)
    wait_writes(0)
    wait_writes(1)


def kernel(ipa_ids, lengths, table):
    ids_t = ipa_ids.T
    out5 = _sc_gather(table, ids_t)
    out = lax.reshape(out5, (_B, _S, _D), dimensions=(2, 4, 0, 1, 3))
    return (out, lengths)
